# baseline (speedup 1.0000x reference)
import dataclasses
import functools

import jax
import jax.numpy as jnp
from jax import lax
from jax.experimental import pallas as pl
from jax.experimental.pallas import tpu as pltpu
from jax.experimental.pallas import tpu_sc as plsc

B = 4096
F = 26
D = 32
L = 16
NC, NS = 2, 16
NW = NC * NS
ITEMS_PER_W = B // NW
CHUNK = 16
NCHUNK = ITEMS_PER_W // CHUNK
CF = CHUNK * F

_MESH = plsc.VectorSubcoreMesh(core_axis_name="c", subcore_axis_name="s")

QR = 250000
W0 = 128
NBLK = 1000000 // W0
BLK_PER_W = -(-NBLK // NW)


def _compiler_params(use_tc_tiling):
    cp = pltpu.CompilerParams()
    fields = pltpu.CompilerParams.__dataclass_fields__
    if "needs_layout_passes" in fields:
        cp = dataclasses.replace(cp, needs_layout_passes=False)
    if "use_tc_tiling_on_sc" in fields:
        cp = dataclasses.replace(cp, use_tc_tiling_on_sc=use_tc_tiling)
    return cp


@functools.partial(
    pl.kernel,
    compiler_params=_compiler_params(True),
    out_type=jax.ShapeDtypeStruct((QR, 128), jnp.float32),
    mesh=_MESH,
    scratch_types=[
        pltpu.VMEM((D, W0), jnp.float32),
        pltpu.VMEM((W0 // 4, 128), jnp.float32),
        pltpu.SemaphoreType.DMA,
    ],
)
def _pack_table(em1t_hbm, tail_hbm, out_hbm, blk_v, out_v, sem):
    wid = lax.axis_index("s") * NC + lax.axis_index("c")
    iota = lax.iota(jnp.int32, L)

    @pl.loop(0, BLK_PER_W)
    def _blk(t):
        j = wid + NW * t

        @pl.when(j < NBLK)
        def _():
            pltpu.sync_copy(em1t_hbm.at[:, pl.ds(j * W0, W0)], blk_v)

            @pl.loop(0, W0 // 4)
            def _row(pp):
                for g in range(8):
                    row = iota + (L if g % 2 else 0)
                    col = jnp.full((L,), 4 * pp + g // 2, dtype=jnp.int32)
                    out_v[pp, pl.ds(g * L, L)] = plsc.load_gather(
                        blk_v, [row, col])

            pltpu.sync_copy(out_v, out_hbm.at[pl.ds(j * (W0 // 4), W0 // 4)])

    @pl.when(wid == 0)
    def _rem():
        pltpu.sync_copy(tail_hbm, out_v.at[pl.ds(0, 16)])
        pltpu.sync_copy(out_v.at[pl.ds(0, 16)],
                        out_hbm.at[pl.ds(NBLK * (W0 // 4), 16)])


@functools.partial(
    pl.kernel,
    compiler_params=_compiler_params(True),
    out_type=jax.ShapeDtypeStruct((B * D,), jnp.float32),
    mesh=_MESH,
    scratch_types=[
        pltpu.VMEM((CF,), jnp.int32),
        pltpu.VMEM((CF,), jnp.int32),
        pltpu.VMEM((CF,), jnp.int32),
        pltpu.VMEM((CF,), jnp.float32),
        pltpu.VMEM((CF, 128), jnp.float32),
        pltpu.VMEM((CHUNK * D,), jnp.float32),
        pltpu.SemaphoreType.DMA,
    ],
)
def _second_order(em1_hbm, idx_hbm, val_hbm, y2_hbm,
                  idx_v, hi_v, lo_v, val_v, rows_v, y2_v, sem1):
    wid = lax.axis_index("s") * NC + lax.axis_index("c")
    iota = lax.iota(jnp.int32, L)
    em1_128 = em1_hbm

    @pl.loop(0, NCHUNK)
    def _chunk(c):
        base = wid * (ITEMS_PER_W * F) + c * CF
        pltpu.sync_copy(idx_hbm.at[pl.ds(base, CF)], idx_v)
        pltpu.sync_copy(val_hbm.at[pl.ds(base, CF)], val_v)

        @pl.loop(0, CF, step=L)
        def _split(j):
            iv = idx_v[pl.ds(j, L)]
            hi_v[pl.ds(j, L)] = iv >> 2
            lo_v[pl.ds(j, L)] = (iv & 3) * D

        pltpu.async_copy(em1_128.at[hi_v], rows_v, sem1).wait()

        @pl.loop(0, CHUNK)
        def _item(i):
            r0 = i * F
            s = [jnp.zeros((L,), jnp.float32) for _ in range(D // L)]
            q = [jnp.zeros((L,), jnp.float32) for _ in range(D // L)]
            for f in range(F):
                r = r0 + f
                rsplat = jnp.full((L,), r, dtype=jnp.int32)
                vb = plsc.load_gather(val_v, [rsplat])
                col = plsc.load_gather(lo_v, [rsplat]) + iota
                for h in range(D // L):
                    e = plsc.load_gather(rows_v, [rsplat, col + h * L])
                    t = e * vb
                    s[h] = s[h] + t
                    q[h] = q[h] + t * t
            for h in range(D // L):
                y2_v[pl.ds(i * D + h * L, L)] = 0.5 * (s[h] * s[h] - q[h])

        ob = wid * (ITEMS_PER_W * D) + c * CHUNK * D
        pltpu.sync_copy(y2_v, y2_hbm.at[pl.ds(ob, CHUNK * D)])


@functools.partial(
    pl.kernel,
    compiler_params=_compiler_params(False),
    out_type=jax.ShapeDtypeStruct((B * F,), jnp.float32),
    mesh=_MESH,
    scratch_types=[
        pltpu.VMEM((CF,), jnp.int32),
        pltpu.VMEM((CF,), jnp.float32),
        pltpu.VMEM((CF,), jnp.float32),
        pltpu.VMEM((CF,), jnp.float32),
        pltpu.SemaphoreType.DMA,
    ],
)
def _first_order(em2_hbm, idx_hbm, val_hbm, y1_hbm,
                 idx_v, val_v, g2_v, y1_v, sem):
    wid = lax.axis_index("s") * NC + lax.axis_index("c")

    @pl.loop(0, NCHUNK)
    def _chunk(c):
        base = wid * (ITEMS_PER_W * F) + c * CF
        pltpu.sync_copy(idx_hbm.at[pl.ds(base, CF)], idx_v)
        pltpu.sync_copy(val_hbm.at[pl.ds(base, CF)], val_v)
        pltpu.async_copy(em2_hbm.at[idx_v], g2_v, sem).wait()

        @pl.loop(0, CF, step=L)
        def _fo(j):
            y1_v[pl.ds(j, L)] = g2_v[pl.ds(j, L)] * val_v[pl.ds(j, L)]

        pltpu.sync_copy(y1_v, y1_hbm.at[pl.ds(base, CF)])


def kernel(feat_index, feat_value, em1_weight, em2_weight):
    idx_flat = feat_index.reshape(-1)
    val_flat = feat_value.reshape(-1)
    em2_flat = em2_weight.reshape(-1)

    em1_tail = em1_weight[NBLK * W0:, :].reshape(16, 128)
    em1_packed = _pack_table(em1_weight.T, em1_tail)
    y2f = _second_order(em1_packed, idx_flat, val_flat)
    y1f = _first_order(em2_flat, idx_flat, val_flat)
    return y1f.reshape(B, F), y2f.reshape(B, D)


# speedup vs baseline: 1.2887x; 1.2887x over previous
import dataclasses
import functools

import jax
import jax.numpy as jnp
from jax import lax
from jax.experimental import pallas as pl
from jax.experimental.pallas import tpu as pltpu
from jax.experimental.pallas import tpu_sc as plsc

B = 4096
F = 26
D = 32
L = 16
NC, NS = 2, 16
NW = NC * NS
ITEMS_PER_W = B // NW
CHUNK = 16
NCHUNK = ITEMS_PER_W // CHUNK
CF = CHUNK * F

_MESH = plsc.VectorSubcoreMesh(core_axis_name="c", subcore_axis_name="s")

QR = 250000
W0 = 512
NBLK = 1000000 // W0
BLK_PER_W = 62
PAIRS = BLK_PER_W // 2


def _compiler_params(use_tc_tiling):
    cp = pltpu.CompilerParams()
    fields = pltpu.CompilerParams.__dataclass_fields__
    if "needs_layout_passes" in fields:
        cp = dataclasses.replace(cp, needs_layout_passes=False)
    if "use_tc_tiling_on_sc" in fields:
        cp = dataclasses.replace(cp, use_tc_tiling_on_sc=use_tc_tiling)
    return cp


@functools.partial(
    pl.kernel,
    compiler_params=_compiler_params(True),
    out_type=jax.ShapeDtypeStruct((QR, 128), jnp.float32),
    mesh=_MESH,
    scratch_types=[
        pltpu.VMEM((D, W0), jnp.float32),
        pltpu.VMEM((D, W0), jnp.float32),
        pltpu.VMEM((W0 // 4, 128), jnp.float32),
        pltpu.VMEM((W0 // 4, 128), jnp.float32),
        pltpu.SemaphoreType.DMA,
        pltpu.SemaphoreType.DMA,
        pltpu.SemaphoreType.DMA,
        pltpu.SemaphoreType.DMA,
    ],
)
def _pack_table(em1t_hbm, tail_hbm, out_hbm,
                blk_a, blk_b, out_a, out_b, sem_a, sem_b, sem_oa, sem_ob):
    wid = lax.axis_index("s") * NC + lax.axis_index("c")
    iota = lax.iota(jnp.int32, L)
    row_lo = iota
    row_hi = iota + L

    def in_slice(j):
        return em1t_hbm.at[:, pl.ds(j * W0, W0)]

    def out_slice(j):
        return out_hbm.at[pl.ds(j * (W0 // 4), W0 // 4)]

    def transpose_block(blk, out_v):
        @pl.loop(0, W0 // 4, step=4)
        def _row(pp0):
            for d in range(4):
                pp = pp0 + d
                colbase = jnp.full((L,), 4 * pp, dtype=jnp.int32)
                for g in range(8):
                    row = row_hi if g % 2 else row_lo
                    out_v[pp, pl.ds(g * L, L)] = plsc.load_gather(
                        blk, [row, colbase + (g // 2)])

    bufs = ((blk_a, sem_a, out_a, sem_oa), (blk_b, sem_b, out_b, sem_ob))

    @pl.when(wid < NBLK)
    def _():
        pltpu.async_copy(in_slice(wid), blk_a, sem_a)

    @pl.loop(0, PAIRS)
    def _pair(u):
        for half in range(2):
            blk, sem, out_v, sem_o = bufs[half]
            nblk, nsem = bufs[1 - half][0], bufs[1 - half][1]
            t = 2 * u + half
            j = wid + NW * t
            jn = j + NW

            @pl.when(jn < NBLK)
            def _(jn=jn, nblk=nblk, nsem=nsem):
                pltpu.async_copy(in_slice(jn), nblk, nsem)

            @pl.when(j < NBLK)
            def _(j=j, blk=blk, sem=sem, out_v=out_v, sem_o=sem_o, u=u,
                  half=half):
                pltpu.make_async_copy(in_slice(j), blk, sem).wait()

                @pl.when(j >= 2 * NW)
                def _():
                    pltpu.make_async_copy(
                        out_v, out_slice(j - 2 * NW), sem_o).wait()

                transpose_block(blk, out_v)
                pltpu.async_copy(out_v, out_slice(j), sem_o)

    j_a = wid + NW * (BLK_PER_W - 2)

    @pl.when(j_a < NBLK)
    def _(j=j_a):
        pltpu.make_async_copy(out_a, out_slice(j), sem_oa).wait()

    j_b1 = wid + NW * (BLK_PER_W - 1)
    j_b3 = wid + NW * (BLK_PER_W - 3)

    @pl.when(j_b1 < NBLK)
    def _(j=j_b1):
        pltpu.make_async_copy(out_b, out_slice(j), sem_ob).wait()

    @pl.when(jnp.logical_and(j_b1 >= NBLK, j_b3 < NBLK))
    def _(j=j_b3):
        pltpu.make_async_copy(out_b, out_slice(j), sem_ob).wait()

    @pl.when(wid == 0)
    def _rem():
        pltpu.sync_copy(tail_hbm, out_a.at[pl.ds(0, 16)])
        pltpu.sync_copy(out_a.at[pl.ds(0, 16)],
                        out_hbm.at[pl.ds(NBLK * (W0 // 4), 16)])


@functools.partial(
    pl.kernel,
    compiler_params=_compiler_params(True),
    out_type=jax.ShapeDtypeStruct((B * D,), jnp.float32),
    mesh=_MESH,
    scratch_types=[
        pltpu.VMEM((CF,), jnp.int32),
        pltpu.VMEM((CF,), jnp.int32),
        pltpu.VMEM((CF,), jnp.int32),
        pltpu.VMEM((CF,), jnp.float32),
        pltpu.VMEM((CF, 128), jnp.float32),
        pltpu.VMEM((CHUNK * D,), jnp.float32),
        pltpu.SemaphoreType.DMA,
    ],
)
def _second_order(em1_hbm, idx_hbm, val_hbm, y2_hbm,
                  idx_v, hi_v, lo_v, val_v, rows_v, y2_v, sem1):
    wid = lax.axis_index("s") * NC + lax.axis_index("c")
    iota = lax.iota(jnp.int32, L)
    em1_128 = em1_hbm

    @pl.loop(0, NCHUNK)
    def _chunk(c):
        base = wid * (ITEMS_PER_W * F) + c * CF
        pltpu.sync_copy(idx_hbm.at[pl.ds(base, CF)], idx_v)
        pltpu.sync_copy(val_hbm.at[pl.ds(base, CF)], val_v)

        @pl.loop(0, CF, step=L)
        def _split(j):
            iv = idx_v[pl.ds(j, L)]
            hi_v[pl.ds(j, L)] = iv >> 2
            lo_v[pl.ds(j, L)] = (iv & 3) * D

        pltpu.async_copy(em1_128.at[hi_v], rows_v, sem1).wait()

        @pl.loop(0, CHUNK)
        def _item(i):
            r0 = i * F
            s = [jnp.zeros((L,), jnp.float32) for _ in range(D // L)]
            q = [jnp.zeros((L,), jnp.float32) for _ in range(D // L)]
            for f in range(F):
                r = r0 + f
                rsplat = jnp.full((L,), r, dtype=jnp.int32)
                vb = plsc.load_gather(val_v, [rsplat])
                col = plsc.load_gather(lo_v, [rsplat]) + iota
                for h in range(D // L):
                    e = plsc.load_gather(rows_v, [rsplat, col + h * L])
                    t = e * vb
                    s[h] = s[h] + t
                    q[h] = q[h] + t * t
            for h in range(D // L):
                y2_v[pl.ds(i * D + h * L, L)] = 0.5 * (s[h] * s[h] - q[h])

        ob = wid * (ITEMS_PER_W * D) + c * CHUNK * D
        pltpu.sync_copy(y2_v, y2_hbm.at[pl.ds(ob, CHUNK * D)])


@functools.partial(
    pl.kernel,
    compiler_params=_compiler_params(False),
    out_type=jax.ShapeDtypeStruct((B * F,), jnp.float32),
    mesh=_MESH,
    scratch_types=[
        pltpu.VMEM((CF,), jnp.int32),
        pltpu.VMEM((CF,), jnp.float32),
        pltpu.VMEM((CF,), jnp.float32),
        pltpu.VMEM((CF,), jnp.float32),
        pltpu.SemaphoreType.DMA,
    ],
)
def _first_order(em2_hbm, idx_hbm, val_hbm, y1_hbm,
                 idx_v, val_v, g2_v, y1_v, sem):
    wid = lax.axis_index("s") * NC + lax.axis_index("c")

    @pl.loop(0, NCHUNK)
    def _chunk(c):
        base = wid * (ITEMS_PER_W * F) + c * CF
        pltpu.sync_copy(idx_hbm.at[pl.ds(base, CF)], idx_v)
        pltpu.sync_copy(val_hbm.at[pl.ds(base, CF)], val_v)
        pltpu.async_copy(em2_hbm.at[idx_v], g2_v, sem).wait()

        @pl.loop(0, CF, step=L)
        def _fo(j):
            y1_v[pl.ds(j, L)] = g2_v[pl.ds(j, L)] * val_v[pl.ds(j, L)]

        pltpu.sync_copy(y1_v, y1_hbm.at[pl.ds(base, CF)])


def kernel(feat_index, feat_value, em1_weight, em2_weight):
    idx_flat = feat_index.reshape(-1)
    val_flat = feat_value.reshape(-1)
    em2_flat = em2_weight.reshape(-1)

    em1_tail = em1_weight[NBLK * W0:, :].reshape(16, 128)
    em1_packed = _pack_table(em1_weight.T, em1_tail)
    y2f = _second_order(em1_packed, idx_flat, val_flat)
    y1f = _first_order(em2_flat, idx_flat, val_flat)
    return y1f.reshape(B, F), y2f.reshape(B, D)


# speedup vs baseline: 2.0455x; 1.5873x over previous
import dataclasses
import functools

import jax
import jax.numpy as jnp
from jax import lax
from jax.experimental import pallas as pl
from jax.experimental.pallas import tpu as pltpu
from jax.experimental.pallas import tpu_sc as plsc

B = 4096
F = 26
D = 32
L = 16
NC, NS = 2, 16
NW = NC * NS
ITEMS_PER_W = B // NW
CHUNK = 16
NCHUNK = ITEMS_PER_W // CHUNK
CF = CHUNK * F


def kernel(feat_index, feat_value, em1_weight, em2_weight):
    idx_flat = feat_index.reshape(-1)
    val_flat = feat_value.reshape(-1)

    mesh = plsc.VectorSubcoreMesh(core_axis_name="c", subcore_axis_name="s")

    cp = pltpu.CompilerParams()
    if "needs_layout_passes" in pltpu.CompilerParams.__dataclass_fields__:
        cp = dataclasses.replace(cp, needs_layout_passes=False)
    if "use_tc_tiling_on_sc" in pltpu.CompilerParams.__dataclass_fields__:
        cp = dataclasses.replace(cp, use_tc_tiling_on_sc=False)

    @functools.partial(
        pl.kernel,
        compiler_params=cp,
        out_type=(
            jax.ShapeDtypeStruct((B * F,), jnp.float32),
            jax.ShapeDtypeStruct((B * D,), jnp.float32),
        ),
        mesh=mesh,
        scratch_types=[
            pltpu.VMEM((CF,), jnp.int32),
            pltpu.VMEM((CF,), jnp.float32),
            pltpu.VMEM((CF, D), jnp.float32),
            pltpu.VMEM((CF,), jnp.float32),
            pltpu.VMEM((CF,), jnp.float32),
            pltpu.VMEM((CHUNK * D,), jnp.float32),
            pltpu.SemaphoreType.DMA,
            pltpu.SemaphoreType.DMA,
        ],
    )
    def fm_kernel(em1_hbm, em2_hbm, idx_hbm, val_hbm, y1_hbm, y2_hbm,
                  idx_v, val_v, rows_v, g2_v, y1_v, y2_v, sem1, sem2):
        wid = lax.axis_index("s") * NC + lax.axis_index("c")

        @pl.loop(0, NCHUNK)
        def _chunk(c):
            base = wid * (ITEMS_PER_W * F) + c * CF
            pltpu.sync_copy(idx_hbm.at[pl.ds(base, CF)], idx_v)
            pltpu.sync_copy(val_hbm.at[pl.ds(base, CF)], val_v)
            cp1 = pltpu.async_copy(em1_hbm.at[idx_v], rows_v, sem1)
            cp2 = pltpu.async_copy(em2_hbm.at[idx_v], g2_v, sem2)
            cp1.wait()
            cp2.wait()

            @pl.loop(0, CHUNK)
            def _item(i):
                r0 = i * F
                s = [jnp.zeros((L,), jnp.float32) for _ in range(D // L)]
                q = [jnp.zeros((L,), jnp.float32) for _ in range(D // L)]
                for f in range(F):
                    r = r0 + f
                    vb = plsc.load_gather(
                        val_v, [jnp.full((L,), r, dtype=jnp.int32)])
                    for h in range(D // L):
                        e = rows_v[r, pl.ds(h * L, L)]
                        t = e * vb
                        s[h] = s[h] + t
                        q[h] = q[h] + t * t
                for h in range(D // L):
                    y2_v[pl.ds(i * D + h * L, L)] = 0.5 * (s[h] * s[h] - q[h])

            @pl.loop(0, CF, step=L)
            def _fo(j):
                y1_v[pl.ds(j, L)] = g2_v[pl.ds(j, L)] * val_v[pl.ds(j, L)]

            pltpu.sync_copy(y1_v, y1_hbm.at[pl.ds(base, CF)])
            ob = wid * (ITEMS_PER_W * D) + c * CHUNK * D
            pltpu.sync_copy(y2_v, y2_hbm.at[pl.ds(ob, CHUNK * D)])

    y1f, y2f = fm_kernel(em1_weight, em2_weight.reshape(-1), idx_flat, val_flat)
    return y1f.reshape(B, F), y2f.reshape(B, D)
